# baseline (speedup 1.0000x reference)
import functools

import jax
import jax.numpy as jnp
from jax import lax
from jax.experimental import pallas as pl
from jax.experimental.pallas import tpu as pltpu
from jax.experimental.pallas import tpu_sc as plsc

ROWS = 4096
COLS = 100000
NSEL = 256

NC = 2
NS = 16
NW = NC * NS

ROWS_PER_W = ROWS // NW
ELEMS_PER_W = ROWS_PER_W * NSEL
CHUNK = 128
NCHUNK = ELEMS_PER_W // CHUNK
INFLIGHT = 8


def _gather_body(flat_hbm, map_hbm, out_hbm, m_v, idx_v, dat_v, sem):
    c_id = lax.axis_index("c")
    s_id = lax.axis_index("s")
    wid = s_id * NC + c_id
    base_row = wid * ROWS_PER_W

    pltpu.sync_copy(map_hbm, m_v)

    def idx_row(r, carry):
        rowscale = jnp.full((16,), (base_row + r) * COLS, dtype=jnp.int32)
        e0 = r * NSEL
        for u in range(NSEL // 16):
            idx_v[pl.ds(e0 + u * 16, 16)] = m_v[pl.ds(u * 16, 16)] + rowscale
        return carry

    lax.fori_loop(0, ROWS_PER_W, idx_row, 0)

    def gather_group(g, carry):
        cps = []
        for u in range(INFLIGHT):
            ch = g * INFLIGHT + u
            cps.append(
                pltpu.async_copy(
                    flat_hbm.at[idx_v.at[pl.ds(ch * CHUNK, CHUNK)]],
                    dat_v.at[pl.ds(ch * CHUNK, CHUNK)],
                    sem,
                )
            )
        for cp in cps:
            cp.wait()
        return carry

    lax.fori_loop(0, NCHUNK // INFLIGHT, gather_group, 0)

    pltpu.sync_copy(dat_v, out_hbm.at[pl.ds(wid * ELEMS_PER_W, ELEMS_PER_W)])


_sc_gather = pl.kernel(
    _gather_body,
    out_type=jax.ShapeDtypeStruct((ROWS * NSEL,), jnp.float32),
    mesh=plsc.VectorSubcoreMesh(
        core_axis_name="c", subcore_axis_name="s", num_cores=NC, num_subcores=NS
    ),
    scratch_types=[
        pltpu.VMEM((NSEL,), jnp.int32),
        pltpu.VMEM((ELEMS_PER_W,), jnp.int32),
        pltpu.VMEM((ELEMS_PER_W,), jnp.float32),
        pltpu.SemaphoreType.DMA,
    ],
)


@jax.jit
def kernel(logits, mapping_sequence):
    flat = logits.reshape(-1)
    out = _sc_gather(flat, mapping_sequence.astype(jnp.int32))
    return out.reshape(ROWS, NSEL)


# speedup vs baseline: 1.0027x; 1.0027x over previous
import functools

import jax
import jax.numpy as jnp
from jax import lax
from jax.experimental import pallas as pl
from jax.experimental.pallas import tpu as pltpu
from jax.experimental.pallas import tpu_sc as plsc

ROWS = 4096
COLS = 100000
NSEL = 256

NC = 2
NS = 16
NW = NC * NS

ROWS_PER_W = ROWS // NW
ELEMS_PER_W = ROWS_PER_W * NSEL
CHUNK = 128
NCHUNK = ELEMS_PER_W // CHUNK
INFLIGHT = 8


def _gather_body(flat_hbm, map_hbm, out_hbm, m_v, idx_v, dat_v, sem):
    c_id = lax.axis_index("c")
    s_id = lax.axis_index("s")
    wid = s_id * NC + c_id
    base_row = wid * ROWS_PER_W

    pltpu.sync_copy(map_hbm, m_v)

    def idx_row(r, carry):
        rowscale = jnp.full((16,), (base_row + r) * COLS, dtype=jnp.int32)
        e0 = r * NSEL
        for u in range(NSEL // 16):
            idx_v[pl.ds(e0 + u * 16, 16)] = m_v[pl.ds(u * 16, 16)] + rowscale
        return carry

    lax.fori_loop(0, ROWS_PER_W, idx_row, 0)

    pltpu.async_copy(flat_hbm.at[idx_v], dat_v, sem).wait()

    pltpu.sync_copy(dat_v, out_hbm.at[pl.ds(wid * ELEMS_PER_W, ELEMS_PER_W)])


_sc_gather = pl.kernel(
    _gather_body,
    out_type=jax.ShapeDtypeStruct((ROWS * NSEL,), jnp.float32),
    mesh=plsc.VectorSubcoreMesh(
        core_axis_name="c", subcore_axis_name="s", num_cores=NC, num_subcores=NS
    ),
    scratch_types=[
        pltpu.VMEM((NSEL,), jnp.int32),
        pltpu.VMEM((ELEMS_PER_W,), jnp.int32),
        pltpu.VMEM((ELEMS_PER_W,), jnp.float32),
        pltpu.SemaphoreType.DMA,
    ],
)


@jax.jit
def kernel(logits, mapping_sequence):
    flat = logits.reshape(-1)
    out = _sc_gather(flat, mapping_sequence.astype(jnp.int32))
    return out.reshape(ROWS, NSEL)


# speedup vs baseline: 1.9210x; 1.9159x over previous
import functools

import jax
import jax.numpy as jnp
from jax import lax
from jax.experimental import pallas as pl
from jax.experimental.pallas import tpu as pltpu
from jax.experimental.pallas import tpu_sc as plsc

ROWS = 4096
COLS = 100000
NSEL = 256
LANE = 128

NC = 2
NS = 16
NW = NC * NS

ROWS_PER_W = ROWS // NW
ELEMS_PER_W = ROWS_PER_W * NSEL


def _gather_body(logits_hbm, map_hbm, out_hbm, m_v, slab, dat_v, sem):
    c_id = lax.axis_index("c")
    s_id = lax.axis_index("s")
    wid = s_id * NC + c_id
    base_row = pl.multiple_of(wid * ROWS_PER_W, 8)

    pltpu.sync_copy(map_hbm, m_v)

    iota16 = lax.iota(jnp.int32, 16)
    iota_rows = iota16 * NSEL

    def do_col(j, carry):
        mvec = m_v[pl.ds((j >> 4) << 4, 16)]
        mj = jnp.sum(jnp.where(iota16 == (j & 15), mvec, 0))
        col0 = pl.multiple_of((mj >> 7) << 7, LANE)
        pltpu.async_copy(
            logits_hbm.at[pl.ds(base_row, ROWS_PER_W), pl.ds(col0, LANE)],
            slab,
            sem,
        ).wait()

        lane = jnp.full((16,), mj & (LANE - 1), dtype=jnp.int32)

        def rowvec(v, c):
            i0 = v * 16
            vals = plsc.load_gather(slab, [iota16 + i0, lane])
            plsc.store_scatter(dat_v, [iota_rows + (i0 * NSEL + j)], vals)
            return c

        lax.fori_loop(0, ROWS_PER_W // 16, rowvec, 0)
        return carry

    lax.fori_loop(0, NSEL, do_col, 0)

    pltpu.sync_copy(dat_v, out_hbm.at[pl.ds(wid * ELEMS_PER_W, ELEMS_PER_W)])


_sc_gather = pl.kernel(
    _gather_body,
    out_type=jax.ShapeDtypeStruct((ROWS * NSEL,), jnp.float32),
    mesh=plsc.VectorSubcoreMesh(
        core_axis_name="c", subcore_axis_name="s", num_cores=NC, num_subcores=NS
    ),
    compiler_params=pltpu.CompilerParams(needs_layout_passes=False),
    scratch_types=[
        pltpu.VMEM((NSEL,), jnp.int32),
        pltpu.VMEM((ROWS_PER_W, LANE), jnp.float32),
        pltpu.VMEM((ELEMS_PER_W,), jnp.float32),
        pltpu.SemaphoreType.DMA,
    ],
)


@jax.jit
def kernel(logits, mapping_sequence):
    out = _sc_gather(logits, mapping_sequence.astype(jnp.int32))
    return out.reshape(ROWS, NSEL)


# speedup vs baseline: 2.1551x; 1.1219x over previous
import functools

import jax
import jax.numpy as jnp
from jax import lax
from jax.experimental import pallas as pl
from jax.experimental.pallas import tpu as pltpu
from jax.experimental.pallas import tpu_sc as plsc

ROWS = 4096
COLS = 100000
NSEL = 256
LANE = 128

NC = 2
NS = 16
NW = NC * NS

ROWS_PER_W = ROWS // NW
ELEMS_PER_W = ROWS_PER_W * NSEL
HALF = ROWS_PER_W // 2
DEPTH = 8


def _gather_body(logits_hbm, map_hbm, out_hbm, m_v, slabs, dat_v, sems):
    c_id = lax.axis_index("c")
    s_id = lax.axis_index("s")
    wid = s_id * NC + c_id
    base_row = pl.multiple_of(wid * ROWS_PER_W, 8)

    pltpu.sync_copy(map_hbm, m_v)

    iota16 = lax.iota(jnp.int32, 16)
    iota_rows = iota16 * NSEL

    def col_of(j):
        mvec = m_v[pl.ds((j >> 4) << 4, 16)]
        return jnp.sum(jnp.where(iota16 == (j & 15), mvec, 0))

    def do_half(h, carry):
        row0 = pl.multiple_of(base_row + h * HALF, 8)

        def fire(j):
            mj = col_of(j)
            col0 = pl.multiple_of((mj >> 7) << 7, LANE)
            slot = j % DEPTH
            pltpu.async_copy(
                logits_hbm.at[pl.ds(row0, HALF), pl.ds(col0, LANE)],
                slabs.at[slot],
                sems.at[slot],
            )

        def wait(j):
            slot = j % DEPTH
            pltpu.make_async_copy(
                logits_hbm.at[pl.ds(0, HALF), pl.ds(0, LANE)],
                slabs.at[slot],
                sems.at[slot],
            ).wait()

        def prologue(j, c):
            fire(j)
            return c

        lax.fori_loop(0, DEPTH - 1, prologue, 0)

        def do_col(j, c):
            @pl.when(j + DEPTH - 1 < NSEL)
            def _():
                fire(j + DEPTH - 1)

            wait(j)
            slot = j % DEPTH
            mj = col_of(j)
            lane = jnp.full((16,), mj & (LANE - 1), dtype=jnp.int32)
            dbase = h * HALF * NSEL + j

            def rowvec(v, c2):
                i0 = v * 16
                vals = plsc.load_gather(slabs.at[slot], [iota16 + i0, lane])
                plsc.store_scatter(
                    dat_v, [iota_rows + (i0 * NSEL + dbase)], vals
                )
                return c2

            lax.fori_loop(0, HALF // 16, rowvec, 0)
            return c

        lax.fori_loop(0, NSEL, do_col, 0)
        return carry

    lax.fori_loop(0, 2, do_half, 0)

    pltpu.sync_copy(dat_v, out_hbm.at[pl.ds(wid * ELEMS_PER_W, ELEMS_PER_W)])


_sc_gather = pl.kernel(
    _gather_body,
    out_type=jax.ShapeDtypeStruct((ROWS * NSEL,), jnp.float32),
    mesh=plsc.VectorSubcoreMesh(
        core_axis_name="c", subcore_axis_name="s", num_cores=NC, num_subcores=NS
    ),
    compiler_params=pltpu.CompilerParams(needs_layout_passes=False),
    scratch_types=[
        pltpu.VMEM((NSEL,), jnp.int32),
        pltpu.VMEM((DEPTH, HALF, LANE), jnp.float32),
        pltpu.VMEM((ELEMS_PER_W,), jnp.float32),
        pltpu.SemaphoreType.DMA((DEPTH,)),
    ],
)


@jax.jit
def kernel(logits, mapping_sequence):
    out = _sc_gather(logits, mapping_sequence.astype(jnp.int32))
    return out.reshape(ROWS, NSEL)
